# Initial kernel scaffold; baseline (speedup 1.0000x reference)
import jax
import jax.numpy as jnp
from jax.experimental import pallas as pl


def kernel(dense_features, sparse_features, emb, bW0, bb0, bg0, be0, bW1, bb1, bg1, be1, bW2, bb2, bg2, be2, tW0, tb0, tg0, te0, tW1, tb1, tg1, te1, tW2, tb2):
    raise NotImplementedError("write your pallas kernel here")


# speedup vs baseline: 3.7166x; 3.7166x over previous
import functools

import numpy as np
import jax
import jax.numpy as jnp
from jax import lax
from jax.experimental import pallas as pl
from jax.experimental.pallas import tpu as pltpu
from jax.experimental.pallas import tpu_sc as plsc

B = 1024
NDENSE = 13
NFIELDS = 26
VOCAB = 100000
EDIM = 16
D_INT = 432
NPAIR = D_INT // 2
PW = 440
OUT_INT = D_INT * (D_INT + 1) // 2
ZW = 768
EPS = 1e-5

SC_CHUNK = 104
SC_NC = 2
SC_NS = 16
SC_NW = SC_NC * SC_NS


def _pair_index_map() -> np.ndarray:
    row_start = np.array(
        [i * D_INT - (i * (i - 1)) // 2 for i in range(D_INT)], dtype=np.int64)
    idx = np.full((NPAIR, PW), OUT_INT, dtype=np.int32)
    for p in range(NPAIR):
        i2 = D_INT - 1 - p
        n1 = D_INT - p
        n2 = p + 1
        idx[p, :n1] = row_start[p] + np.arange(n1)
        idx[p, n1:n1 + n2] = row_start[i2] + np.arange(n2)
    return idx


_PAIR_IDX = _pair_index_map()


def _embed_gather(table, idx3):
    nchunk = idx3.shape[1]
    b_per_w = nchunk * SC_CHUNK
    n = SC_NW * b_per_w
    mesh = plsc.VectorSubcoreMesh(core_axis_name="c", subcore_axis_name="s")

    @functools.partial(
        pl.kernel, mesh=mesh,
        out_type=jax.ShapeDtypeStruct((n, EDIM), jnp.float32),
        compiler_params=pltpu.CompilerParams(use_tc_tiling_on_sc=False),
        scratch_types=[
            pltpu.VMEM((nchunk, SC_CHUNK), jnp.int32),
            pltpu.VMEM((b_per_w, EDIM), jnp.float32),
            pltpu.SemaphoreType.DMA,
        ],
    )
    def k(table_hbm, idx_hbm, out_hbm, idx_v, rows_v, sem):
        wid = lax.axis_index("s") * SC_NC + lax.axis_index("c")
        base = wid * b_per_w
        pltpu.sync_copy(idx_hbm.at[wid], idx_v)
        copies = []
        for j in range(nchunk):
            copies.append(pltpu.async_copy(
                table_hbm.at[idx_v.at[j]],
                rows_v.at[pl.ds(j * SC_CHUNK, SC_CHUNK)],
                sem))
        for c in copies:
            c.wait()
        pltpu.sync_copy(rows_v, out_hbm.at[pl.ds(base, b_per_w)])

    return k(table, idx3)


def _bn_relu(x, g, b):
    mu = jnp.mean(x, axis=0, keepdims=True)
    var = jnp.mean((x - mu) ** 2, axis=0, keepdims=True)
    return jnp.maximum((x - mu) * lax.rsqrt(var + EPS) * g + b, 0.0)


def _bot_body(dense_ref, w0_ref, g0_ref, e0_ref, w1_ref, g1_ref, e1_ref,
              w2_ref, g2_ref, e2_ref, out_ref):
    f32 = jnp.float32
    x = dense_ref[...]
    h = _bn_relu(jnp.dot(x, w0_ref[...], preferred_element_type=f32),
                 g0_ref[...], e0_ref[...])
    h = _bn_relu(jnp.dot(h, w1_ref[...], preferred_element_type=f32),
                 g1_ref[...], e1_ref[...])
    h = _bn_relu(jnp.dot(h, w2_ref[...], preferred_element_type=f32),
                 g2_ref[...], e2_ref[...])
    out_ref[...] = h


def _bottom_mlp(dense, w0, g0, e0, w1, g1, e1, w2, g2, e2):
    return pl.pallas_call(
        _bot_body,
        out_shape=jax.ShapeDtypeStruct((B, 16), jnp.float32),
    )(dense, w0, g0, e0, w1, g1, e1, w2, g2, e2)


def _top_body(h_ref, e_ref, w0h_ref, wp_ref, tg0_ref, te0_ref,
              tW1_ref, tg1_ref, te1_ref, tW2_ref, tb2_ref, out_ref,
              zpad_ref, zsh_ref, acc_ref):
    f32 = jnp.float32
    i = pl.program_id(0)

    @pl.when(i == 0)
    def _init():
        z = jnp.concatenate([h_ref[...], e_ref[...]], axis=1)
        zpad_ref[...] = jnp.zeros((B, ZW), f32)
        zpad_ref[:, 0:D_INT] = z
        zsh_ref[...] = jnp.zeros((B, PW), f32)
        zsh_ref[:, 1:D_INT + 1] = z
        acc_ref[...] = jnp.dot(h_ref[...], w0h_ref[...],
                               preferred_element_type=f32)

    zpad = zpad_ref[...]
    sr = pltpu.roll(zpad, lax.rem(ZW - i, ZW), 1)
    s1 = sr[:, :PW]
    c1 = s1[:, 0:1]
    zcol = lax.broadcasted_iota(jnp.int32, (1, ZW), 1)
    c2 = jnp.sum(jnp.where(zcol == (D_INT - 1 - i), zpad, 0.0),
                 axis=1, keepdims=True)
    s2 = zsh_ref[...]
    tcol = lax.broadcasted_iota(jnp.int32, (1, PW), 1)
    v = jnp.where(tcol < (D_INT - i), c1 * s1, c2 * s2)
    acc_ref[...] += lax.dot_general(
        v, wp_ref[0], (((1,), (0,)), ((), ())), preferred_element_type=f32)

    @pl.when(i == NPAIR - 1)
    def _fin():
        x = acc_ref[...]
        x = _bn_relu(x, tg0_ref[...], te0_ref[...])
        x = _bn_relu(jnp.dot(x, tW1_ref[...], preferred_element_type=f32),
                     tg1_ref[...], te1_ref[...])
        out_ref[...] = (jnp.dot(x, tW2_ref[...], preferred_element_type=f32)
                        + tb2_ref[...])


def _top_mlp(h, e, w0h, wp, tg0, te0, tW1, tg1, te1, tW2, tb2):
    full = lambda shape: pl.BlockSpec(shape, lambda i: tuple(0 for _ in shape))
    return pl.pallas_call(
        _top_body,
        grid=(NPAIR,),
        in_specs=[
            full((B, 16)),
            full((B, NFIELDS * EDIM)),
            full((16, 256)),
            pl.BlockSpec((1, PW, 256), lambda i: (i, 0, 0)),
            full((1, 256)), full((1, 256)),
            full((256, 128)), full((1, 128)), full((1, 128)),
            full((128, 1)), full((1, 1)),
        ],
        out_specs=full((B, 1)),
        out_shape=jax.ShapeDtypeStruct((B, 1), jnp.float32),
        scratch_shapes=[
            pltpu.VMEM((B, ZW), jnp.float32),
            pltpu.VMEM((B, PW), jnp.float32),
            pltpu.VMEM((B, 256), jnp.float32),
        ],
    )(h, e, w0h, wp, tg0, te0, tW1, tg1, te1, tW2, tb2)


def kernel(dense_features, sparse_features, emb, bW0, bb0, bg0, be0,
           bW1, bb1, bg1, be1, bW2, bb2, bg2, be2,
           tW0, tb0, tg0, te0, tW1, tb1, tg1, te1, tW2, tb2):
    del bb0, bb1, bb2, tb0, tb1

    flat_tables = emb.reshape(NFIELDS * VOCAB, EDIM)
    idx = (sparse_features.astype(jnp.int32)
           + (jnp.arange(NFIELDS, dtype=jnp.int32) * VOCAB)[None, :])
    idx3 = idx.reshape(SC_NW, (B * NFIELDS) // (SC_NW * SC_CHUNK), SC_CHUNK)

    w0h = tW0[:16]
    w_ext = jnp.concatenate(
        [tW0[16:], jnp.zeros((1, 256), jnp.float32)], axis=0)
    wp = jnp.take(w_ext, jnp.asarray(_PAIR_IDX.reshape(-1)), axis=0)
    wp = wp.reshape(NPAIR, PW, 256)

    row2 = lambda a: a.reshape(1, -1)

    e = _embed_gather(flat_tables, idx3)
    h = _bottom_mlp(dense_features, bW0, row2(bg0), row2(be0),
                    bW1, row2(bg1), row2(be1),
                    bW2, row2(bg2), row2(be2))
    e = e.reshape(B, NFIELDS * EDIM)
    out = _top_mlp(h, e, w0h, wp, row2(tg0), row2(te0),
                   tW1, row2(tg1), row2(te1), tW2, row2(tb2))
    return out
